# baseline (speedup 1.0000x reference)
import functools

import jax
import jax.numpy as jnp
from jax import lax
from jax.experimental import pallas as pl
from jax.experimental.pallas import tpu as pltpu
from jax.experimental.pallas import tpu_sc as plsc

B = 16384
D = 64

NC = 2
NS = 16
NW = NC * NS
BPW = B // NW

def _sc_gather_body(u_hbm, i_hbm, gu_tab, gi_tab, mu_tab, mi_tab,
                    gu_out, gi_out, mu_out, mi_out,
                    uidx, iidx, buf0, buf1, buf2,
                    sg0, sg1, sg2, sg3, so0, so1, so2, so3):
    wid = lax.axis_index("s") * NC + lax.axis_index("c")
    base = wid * BPW
    sl = pl.ds(base, BPW)
    pltpu.sync_copy(u_hbm.at[sl], uidx)
    pltpu.sync_copy(i_hbm.at[sl], iidx)
    g0 = pltpu.async_copy(gu_tab.at[uidx], buf0, sg0)
    g1 = pltpu.async_copy(gi_tab.at[iidx], buf1, sg1)
    g2 = pltpu.async_copy(mu_tab.at[uidx], buf2, sg2)
    g0.wait()
    o0 = pltpu.async_copy(buf0, gu_out.at[sl], so0)
    g1.wait()
    o1 = pltpu.async_copy(buf1, gi_out.at[sl], so1)
    o0.wait()
    g3 = pltpu.async_copy(mi_tab.at[iidx], buf0, sg3)
    g2.wait()
    o2 = pltpu.async_copy(buf2, mu_out.at[sl], so2)
    g3.wait()
    o3 = pltpu.async_copy(buf0, mi_out.at[sl], so3)
    o1.wait()
    o2.wait()
    o3.wait()


@functools.cache
def _sc_gather():
    mesh = plsc.VectorSubcoreMesh(
        core_axis_name="c", subcore_axis_name="s", num_cores=NC, num_subcores=NS
    )
    return pl.kernel(
        _sc_gather_body,
        out_type=[jax.ShapeDtypeStruct((B, D), jnp.float32)] * 4,
        mesh=mesh,
        scratch_types=[
            pltpu.VMEM((BPW,), jnp.int32),
            pltpu.VMEM((BPW,), jnp.int32),
            pltpu.VMEM((BPW, D), jnp.float32),
            pltpu.VMEM((BPW, D), jnp.float32),
            pltpu.VMEM((BPW, D), jnp.float32),
        ] + [pltpu.SemaphoreType.DMA] * 8,
        compiler_params=pltpu.CompilerParams(use_tc_tiling_on_sc=False),
    )


def _tc_body(gu_ref, gi_ref, mu_ref, mi_ref,
             w1_ref, b1_ref, w2_ref, b2_ref, w3_ref, b3_ref,
             wp_ref, bp_ref, out_ref):
    w1 = w1_ref[...]
    h = jnp.dot(mu_ref[...], w1[:D], preferred_element_type=jnp.float32)
    h = h + jnp.dot(mi_ref[...], w1[D:], preferred_element_type=jnp.float32)
    h = jnp.maximum(h + b1_ref[...], 0.0)
    h = jnp.maximum(
        jnp.dot(h, w2_ref[...], preferred_element_type=jnp.float32) + b2_ref[...], 0.0)
    h = jnp.maximum(
        jnp.dot(h, w3_ref[...], preferred_element_type=jnp.float32) + b3_ref[...], 0.0)
    g = gu_ref[...] * gi_ref[...]
    wp = wp_ref[...]
    pred = jnp.dot(g, wp[:D], preferred_element_type=jnp.float32)
    pred = pred + jnp.dot(h, wp[D:], preferred_element_type=jnp.float32)
    out_ref[...] = pred + bp_ref[...]


def _tc_dense(gu, gi, mu, mi, W1, b1, W2, b2, W3, b3, Wp, bp):
    R = 2048
    grid = (B // R,)
    row_spec = pl.BlockSpec((R, D), lambda r: (r, 0))

    def full(shape):
        return pl.BlockSpec(shape, lambda r: (0,) * len(shape))

    return pl.pallas_call(
        _tc_body,
        grid=grid,
        in_specs=[
            row_spec, row_spec, row_spec, row_spec,
            full(W1.shape), full((1, b1.shape[0])),
            full(W2.shape), full((1, b2.shape[0])),
            full(W3.shape), full((1, b3.shape[0])),
            full(Wp.shape), full((1, 1)),
        ],
        out_specs=pl.BlockSpec((R, 1), lambda r: (r, 0)),
        out_shape=jax.ShapeDtypeStruct((B, 1), jnp.float32),
    )(gu, gi, mu, mi, W1, b1.reshape(1, -1), W2, b2.reshape(1, -1),
      W3, b3.reshape(1, -1), Wp, bp.reshape(1, 1))


def kernel(u, i, gmf_user_table, gmf_item_table, mlp_user_table, mlp_item_table,
           W1, b1, W2, b2, W3, b3, Wp, bp):
    u = u.astype(jnp.int32)
    i = i.astype(jnp.int32)
    gu, gi, mu, mi = _sc_gather()(
        u, i, gmf_user_table, gmf_item_table, mlp_user_table, mlp_item_table)
    out = _tc_dense(gu, gi, mu, mi, W1, b1, W2, b2, W3, b3, Wp, bp)
    return out[:, 0]
